# baseline (device time: 29449 ns/iter reference)
import jax
import jax.numpy as jnp
from jax import lax
from jax.experimental import pallas as pl
from jax.experimental.pallas import tpu as pltpu

N_DEV = 32


def kernel(x, w_mat):
    m_per, k = x.shape
    _, n = w_mat.shape
    n_per = n // N_DEV

    def body(x_ref, w_ref, out_ref, tiles_ref, ssem, rsem):
        my = lax.axis_index("i")

        barrier_sem = pltpu.get_barrier_semaphore()
        for d in range(1, N_DEV):
            pl.semaphore_signal(
                barrier_sem, inc=1,
                device_id=(lax.rem(my + d, N_DEV),),
                device_id_type=pl.DeviceIdType.MESH,
            )

        y = jnp.maximum(
            jnp.dot(x_ref[...], w_ref[...], preferred_element_type=jnp.float32),
            0.0,
        )
        pl.semaphore_wait(barrier_sem, N_DEV - 1)

        for t in range(N_DEV):
            tiles_ref[t] = y[:, t * n_per:(t + 1) * n_per]
            pltpu.make_async_remote_copy(
                src_ref=tiles_ref.at[t],
                dst_ref=out_ref.at[pl.ds(my * m_per, m_per), :],
                send_sem=ssem,
                recv_sem=rsem,
                device_id=(t,),
                device_id_type=pl.DeviceIdType.MESH,
            ).start()

        agg = out_ref.at[pl.ds(0, N_DEV * m_per), :]
        pltpu.make_async_remote_copy(
            src_ref=agg, dst_ref=agg,
            send_sem=ssem, recv_sem=rsem,
            device_id=(my,),
            device_id_type=pl.DeviceIdType.MESH,
        ).wait_recv()

        agg_s = tiles_ref.at[pl.ds(0, N_DEV)]
        pltpu.make_async_remote_copy(
            src_ref=agg_s, dst_ref=agg_s,
            send_sem=ssem, recv_sem=rsem,
            device_id=(my,),
            device_id_type=pl.DeviceIdType.MESH,
        ).wait_send()

    return pl.pallas_call(
        body,
        out_shape=jax.ShapeDtypeStruct((N_DEV * m_per, n_per), jnp.float32),
        in_specs=[
            pl.BlockSpec(memory_space=pltpu.VMEM),
            pl.BlockSpec(memory_space=pltpu.VMEM),
        ],
        out_specs=pl.BlockSpec(memory_space=pltpu.VMEM),
        scratch_shapes=[
            pltpu.VMEM((N_DEV, m_per, n_per), jnp.float32),
            pltpu.SemaphoreType.DMA,
            pltpu.SemaphoreType.DMA,
        ],
        compiler_params=pltpu.CompilerParams(collective_id=0),
    )(x, w_mat)


# device time: 28112 ns/iter; 1.0476x vs baseline; 1.0476x over previous
import jax
import jax.numpy as jnp
from jax import lax
from jax.experimental import pallas as pl
from jax.experimental.pallas import tpu as pltpu

N_DEV = 32


def kernel(x, w_mat):
    m_per, k = x.shape
    _, n = w_mat.shape
    n_per = n // N_DEV

    def body(x_ref, w_ref, out_ref, tiles_ref, ssem, rsem):
        my = lax.axis_index("i")

        barrier_sem = pltpu.get_barrier_semaphore()
        for d in range(1, N_DEV):
            pl.semaphore_signal(
                barrier_sem, inc=1,
                device_id=(lax.rem(my + d, N_DEV),),
                device_id_type=pl.DeviceIdType.MESH,
            )

        y = jnp.maximum(
            jnp.dot(x_ref[...], w_ref[...], preferred_element_type=jnp.float32),
            0.0,
        )
        for t in range(N_DEV):
            tiles_ref[t] = y[:, t * n_per:(t + 1) * n_per]

        out_ref[pl.ds(my * m_per, m_per), :] = tiles_ref[my]

        pl.semaphore_wait(barrier_sem, N_DEV - 1)

        for d in range(1, N_DEV):
            tgt = lax.rem(my + d, N_DEV)
            pltpu.make_async_remote_copy(
                src_ref=tiles_ref.at[tgt],
                dst_ref=out_ref.at[pl.ds(my * m_per, m_per), :],
                send_sem=ssem,
                recv_sem=rsem,
                device_id=(tgt,),
                device_id_type=pl.DeviceIdType.MESH,
            ).start()

        agg = out_ref.at[pl.ds(0, (N_DEV - 1) * m_per), :]
        pltpu.make_async_remote_copy(
            src_ref=agg, dst_ref=agg,
            send_sem=ssem, recv_sem=rsem,
            device_id=(my,),
            device_id_type=pl.DeviceIdType.MESH,
        ).wait_recv()

        agg_s = tiles_ref.at[pl.ds(0, N_DEV - 1)]
        pltpu.make_async_remote_copy(
            src_ref=agg_s, dst_ref=agg_s,
            send_sem=ssem, recv_sem=rsem,
            device_id=(my,),
            device_id_type=pl.DeviceIdType.MESH,
        ).wait_send()

    return pl.pallas_call(
        body,
        out_shape=jax.ShapeDtypeStruct((N_DEV * m_per, n_per), jnp.float32),
        in_specs=[
            pl.BlockSpec(memory_space=pltpu.VMEM),
            pl.BlockSpec(memory_space=pltpu.VMEM),
        ],
        out_specs=pl.BlockSpec(memory_space=pltpu.VMEM),
        scratch_shapes=[
            pltpu.VMEM((N_DEV, m_per, n_per), jnp.float32),
            pltpu.SemaphoreType.DMA,
            pltpu.SemaphoreType.DMA,
        ],
        compiler_params=pltpu.CompilerParams(collective_id=0),
    )(x, w_mat)
